# Initial kernel scaffold; baseline (speedup 1.0000x reference)
import jax
import jax.numpy as jnp
from jax.experimental import pallas as pl


def kernel(conv_seqs, tree_tokens, C0, C1, C2, C3):
    raise NotImplementedError("write your pallas kernel here")


# speedup vs baseline: 7.6017x; 7.6017x over previous
import functools

import jax
import jax.numpy as jnp
from jax import lax
from jax.experimental import pallas as pl
from jax.experimental.pallas import tpu as pltpu
from jax.experimental.pallas import tpu_sc as plsc

B = 1024
D = 64
HD = 4 * D
N_CONV = 400
N_TREE = 160
N_TOK = N_CONV + N_TREE
N_SEG = 60
SEG_PAD = 64
CHUNK = 112
N_CHUNK = N_TOK // CHUNK

_CHUNK_GROUPS = {
    0: [(14, 0, 0, 8)],
    1: [(14, 0, 14, 8)],
    2: [(14, 0, 28, 8)],
    3: [(8, 0, 42, 8), (3, 64, 50, 16)],
    4: [(7, 0, 53, 16)],
}


def _sc_segment_sums(idx, T):
    info = plsc.get_sparse_core_info()
    nc, ns = info.num_cores, info.num_subcores
    nw = nc * ns
    b_per_w = B // nw

    mesh = plsc.VectorSubcoreMesh(core_axis_name="c", subcore_axis_name="s")

    @functools.partial(
        pl.kernel,
        mesh=mesh,
        out_type=jax.ShapeDtypeStruct((B, SEG_PAD, HD), jnp.float32),
        scratch_types=[
            pltpu.VMEM((N_TOK,), jnp.int32),
            pltpu.VMEM((CHUNK, HD), jnp.float32),
            pltpu.VMEM((CHUNK, HD), jnp.float32),
            pltpu.VMEM((SEG_PAD, HD), jnp.float32),
            pltpu.SemaphoreType.DMA,
            pltpu.SemaphoreType.DMA,
        ],
    )
    def sc_kernel(idx_hbm, t_hbm, out_hbm, idx_v, buf0, buf1, s_v, sem0, sem1):
        wid = lax.axis_index("s") * nc + lax.axis_index("c")
        bufs = (buf0, buf1)
        sems = (sem0, sem1)
        zeros16 = jnp.zeros((16,), jnp.float32)
        for j in range(N_SEG, SEG_PAD):
            for v in range(HD // 16):
                s_v[j, pl.ds(v * 16, 16)] = zeros16

        def gather(c):
            return pltpu.async_copy(
                t_hbm.at[idx_v.at[pl.ds(c * CHUNK, CHUNK)]],
                bufs[c % 2],
                sems[c % 2],
            )

        def reduce_group(buf, n, row0, seg0, seg_len):
            def body(j, _):
                base = row0 + j * seg_len
                seg = seg0 + j
                for v in range(HD // 16):
                    sl = pl.ds(v * 16, 16)
                    acc = buf[base, sl]
                    for k in range(1, seg_len):
                        acc = acc + buf[base + k, sl]
                    s_v[seg, sl] = acc
                return 0

            lax.fori_loop(0, n, body, 0)

        def do_row(i, _):
            b = wid * b_per_w + i
            pltpu.sync_copy(idx_hbm.at[b], idx_v)
            cur = gather(0)
            for c in range(N_CHUNK):
                nxt = gather(c + 1) if c + 1 < N_CHUNK else None
                cur.wait()
                for n, row0, seg0, seg_len in _CHUNK_GROUPS[c]:
                    reduce_group(bufs[c % 2], n, row0, seg0, seg_len)
                cur = nxt
            pltpu.sync_copy(s_v, out_hbm.at[b])
            return 0

        lax.fori_loop(0, b_per_w, do_row, 0)

    return sc_kernel(idx, T)


def _chain_body(s_ref, o_ref):
    S = s_ref[...]
    neg = jnp.float32(-1e30)
    jmask = lax.broadcasted_iota(jnp.int32, (1, SEG_PAD), 1) < N_SEG
    u = jnp.sum(S[:, :, D : 2 * D], axis=1) * jnp.float32(1.0 / N_SEG)
    for h in (1, 2):
        sh = S[:, :, h * D : (h + 1) * D]
        sn = S[:, :, (h + 1) * D : (h + 2) * D]
        dots = jnp.sum(sh * u[:, None, :], axis=2)
        dots = jnp.where(jmask, dots, neg)
        m = jnp.max(dots, axis=1, keepdims=True)
        e = jnp.exp(dots - m)
        p = e / jnp.sum(e, axis=1, keepdims=True)
        u = u + jnp.sum(sn * p[:, :, None], axis=1)
    o_ref[...] = u


def _tc_chain(S):
    blk = 128
    return pl.pallas_call(
        _chain_body,
        grid=(B // blk,),
        in_specs=[pl.BlockSpec((blk, SEG_PAD, HD), lambda i: (i, 0, 0))],
        out_specs=pl.BlockSpec((blk, D), lambda i: (i, 0)),
        out_shape=jax.ShapeDtypeStruct((B, D), jnp.float32),
    )(S)


def kernel(conv_seqs, tree_tokens, C0, C1, C2, C3):
    idx = jnp.concatenate(
        [conv_seqs.reshape(B, N_CONV), tree_tokens.reshape(B, N_TREE)], axis=1
    )
    T = jnp.concatenate([C0, C1, C2, C3], axis=1)
    S = _sc_segment_sums(idx, T)
    return _tc_chain(S)
